# baseline (speedup 1.0000x reference)
import jax
import jax.numpy as jnp
from jax import lax
from jax.experimental import pallas as pl
from jax.experimental.pallas import tpu as pltpu
from jax.experimental.pallas import tpu_sc as plsc

B, T, D, E = 32, 4096, 768, 64
K = 64
KEV = 32
NV = T // 16
SIGN = -2**31
BB = 8


def _unrolled_fori(n, unroll, body, carry):
    def outer(o, c):
        for u in range(unroll):
            c = body(o * unroll + u, c)
        return c
    return lax.fori_loop(0, n // unroll, outer, carry)


def _sc_body(ev_hbm, seq_hbm, tt_hbm,
             times_hbm, gseq_hbm, gte_hbm,
             raw_v, keys_v, comp_v, chosen_v, idxf_v,
             row_seq, row_te, gsems, wsem):
    w = lax.axis_index("s") * 2 + lax.axis_index("c")

    pltpu.sync_copy(ev_hbm.at[w], raw_v)

    def kt_body(i, car):
        m1, m2 = car
        v = raw_v[pl.ds(i * 16, 16)]
        bits = lax.bitcast_convert_type(v, jnp.int32)
        ks = jnp.where(bits >= 0, bits, bits ^ jnp.int32(0x7FFFFFFF))
        keys_v[pl.ds(i * 16, 16)] = ks
        return jnp.maximum(m1, ks), jnp.maximum(m2, jnp.minimum(m1, ks))

    sentinel = jnp.full((16,), SIGN, jnp.int32)
    m1, m2 = _unrolled_fori(NV, 16, kt_body, (sentinel, sentinel))
    thr_lb = jnp.min(m2)

    def cp_body(i, selc):
        ks = keys_v[pl.ds(i * 16, 16)]
        m = ks >= thr_lb

        @pl.when(jnp.any(m))
        def _():
            mi = m.astype(jnp.int32)
            rank = selc + plsc.cumsum(mi) - mi
            plsc.store_scatter(comp_v, [rank], ks, mask=m)

        return selc + plsc.all_reduce_population_count(m)

    selc = _unrolled_fori(NV, 8, cp_body, jnp.zeros((16,), jnp.int32))
    csize = jnp.max(selc)
    nv2 = (csize + 15) // 16

    def count_subset(pred):
        def cnt(i, acc):
            ks = comp_v[pl.ds(i * 16, 16)]
            lanes_ok = (lax.iota(jnp.int32, 16) + i * 16) < csize
            return acc + (pred(ks) & lanes_ok).astype(jnp.int32)
        return jnp.sum(lax.fori_loop(0, nv2, cnt, jnp.zeros((16,), jnp.int32)))

    def bit_body(bi, prefix_b):
        cand_b = prefix_b | (jnp.int32(1) << (31 - bi))
        cand_s = cand_b ^ SIGN
        cnt = count_subset(lambda ks: ks >= cand_s)
        return jnp.where(cnt >= KEV, cand_b, prefix_b)

    prefix_b = lax.fori_loop(0, 32, bit_body, jnp.int32(0))
    thr = prefix_b ^ SIGN
    cnt_gt = count_subset(lambda ks: ks > thr)
    need = KEV - cnt_gt

    def fin_body(i, carries):
        selc, eqc = carries
        ks = keys_v[pl.ds(i * 16, 16)]
        t = lax.iota(jnp.int32, 16) + i * 16
        gt = ks > thr
        eq = ks == thr

        def with_ties():
            eqi = eq.astype(jnp.int32)
            eq_rank = eqc + plsc.cumsum(eqi) - eqi
            return gt | (eq & (eq_rank < need))

        sel = lax.cond(jnp.any(eq), with_ties, lambda: gt)
        m = sel | (lax.rem(t, 65) == 0)

        @pl.when(jnp.any(m))
        def _():
            mi = m.astype(jnp.int32)
            rank = selc + plsc.cumsum(mi) - mi
            plsc.store_scatter(chosen_v, [rank], t, mask=m & (rank < K))

        return (selc + plsc.all_reduce_population_count(m),
                eqc + plsc.all_reduce_population_count(eq))

    z16 = jnp.zeros((16,), jnp.int32)
    _unrolled_fori(NV, 8, fin_body, (z16, z16))

    pltpu.sync_copy(chosen_v, times_hbm.at[pl.ds(w * K, K)])
    for j in range(K // 16):
        idxf_v[pl.ds(j * 16, 16)] = chosen_v[pl.ds(j * 16, 16)] + w * T

    CH = 16
    seq_in = [pltpu.async_copy(seq_hbm.at[idxf_v.at[pl.ds(c * CH, CH)]],
                               row_seq.at[pl.ds(c * CH, CH)], gsems[c])
              for c in range(K // CH)]
    te_in = pltpu.async_copy(tt_hbm.at[chosen_v], row_te, gsems[K // CH])
    outs = []
    for c in range(K // CH):
        seq_in[c].wait()
        outs.append(pltpu.async_copy(row_seq.at[pl.ds(c * CH, CH)],
                                     gseq_hbm.at[w].at[pl.ds(c * CH, CH)],
                                     wsem))
    te_in.wait()
    outs.append(pltpu.async_copy(row_te, gte_hbm.at[w], wsem))
    for o in outs:
        o.wait()


def _sc_select_gather(event_scores, seq2, time_table):
    return pl.kernel(
        _sc_body,
        out_type=(
            jax.ShapeDtypeStruct((B * K,), jnp.int32),
            jax.ShapeDtypeStruct((B, K, D), jnp.float32),
            jax.ShapeDtypeStruct((B, K, D), jnp.float32),
        ),
        mesh=plsc.VectorSubcoreMesh(core_axis_name="c", subcore_axis_name="s",
                                    num_cores=2, num_subcores=16),
        compiler_params=pltpu.CompilerParams(needs_layout_passes=False),
        scratch_types=[
            pltpu.VMEM((T,), jnp.float32),
            pltpu.VMEM((T,), jnp.int32),
            pltpu.VMEM((T,), jnp.int32),
            pltpu.VMEM((K,), jnp.int32),
            pltpu.VMEM((K,), jnp.int32),
            pltpu.VMEM((K, D), jnp.float32),
            pltpu.VMEM((K, D), jnp.float32),
            [pltpu.SemaphoreType.DMA] * 5,
            pltpu.SemaphoreType.DMA,
        ],
    )(event_scores, seq2, time_table)


def _tc_body(times_ref, gseq_ref, hl_ref, gte_ref, wt_ref, b_ref,
             out_ref, hold_ref, hl_scr):
    g = pl.program_id(0)
    for j in range(BB):
        for r in range(K):
            idx = times_ref[g * BB * K + j * K + r]
            hl_scr[pl.ds(j * K + r, 1), :] = hl_ref[j, pl.ds(idx, 1), :]
    hl = hl_scr[...]
    hold_ref[...] = hl.reshape(BB, K, E)
    mx = jnp.max(hl, axis=-1, keepdims=True)
    ex = jnp.exp(hl - mx)
    sm = ex / jnp.sum(ex, axis=-1, keepdims=True)
    x = gseq_ref[...].reshape(BB * K, D)
    acc = jnp.dot(x, wt_ref[:D], preferred_element_type=jnp.float32)
    acc = acc + jnp.dot(sm, wt_ref[D:], preferred_element_type=jnp.float32)
    acc = acc + b_ref[...] + gte_ref[...].reshape(BB * K, D)
    out_ref[...] = acc.reshape(BB, K, D)


def _tc_project(times1, gseq, hl, gte, WT, b2):
    return pl.pallas_call(
        _tc_body,
        grid=(B // BB,),
        in_specs=[
            pl.BlockSpec(memory_space=pltpu.SMEM),
            pl.BlockSpec((BB, K, D), lambda b: (b, 0, 0)),
            pl.BlockSpec((BB, T, E), lambda b: (b, 0, 0)),
            pl.BlockSpec((BB, K, D), lambda b: (b, 0, 0)),
            pl.BlockSpec((D + E, D), lambda b: (0, 0)),
            pl.BlockSpec((1, D), lambda b: (0, 0)),
        ],
        out_specs=(
            pl.BlockSpec((BB, K, D), lambda b: (b, 0, 0)),
            pl.BlockSpec((BB, K, E), lambda b: (b, 0, 0)),
        ),
        out_shape=(
            jax.ShapeDtypeStruct((B, K, D), jnp.float32),
            jax.ShapeDtypeStruct((B, K, E), jnp.float32),
        ),
        scratch_shapes=[pltpu.VMEM((BB * K, E), jnp.float32)],
    )(times1, gseq, hl, gte, WT, b2)


def kernel(sequence, holder_logits, event_scores, W, b_lin, time_table):
    seq2 = sequence.reshape(B * T, D)
    times1, gseq, gte = _sc_select_gather(event_scores, seq2, time_table)
    entries, holders = _tc_project(times1, gseq, holder_logits, gte, W.T,
                                   b_lin.reshape(1, D))
    mask = jnp.ones((B, K), jnp.bool_)
    return entries, mask, times1.reshape(B, K), holders


# speedup vs baseline: 1.0043x; 1.0043x over previous
import jax
import jax.numpy as jnp
from jax import lax
from jax.experimental import pallas as pl
from jax.experimental.pallas import tpu as pltpu
from jax.experimental.pallas import tpu_sc as plsc

B, T, D, E = 32, 4096, 768, 64
K = 64
KEV = 32
NV = T // 16
SIGN = -2**31
BB = 4


def _unrolled_fori(n, unroll, body, carry):
    def outer(o, c):
        for u in range(unroll):
            c = body(o * unroll + u, c)
        return c
    return lax.fori_loop(0, n // unroll, outer, carry)


def _sc_body(ev_hbm, seq_hbm, tt_hbm,
             times_hbm, gseq_hbm, gte_hbm,
             raw_v, keys_v, comp_v, chosen_v, idxf_v,
             row_seq, row_te, gsems, wsem):
    w = lax.axis_index("s") * 2 + lax.axis_index("c")

    pltpu.sync_copy(ev_hbm.at[w], raw_v)

    def kt_body(i, car):
        m1, m2 = car
        v = raw_v[pl.ds(i * 16, 16)]
        bits = lax.bitcast_convert_type(v, jnp.int32)
        ks = jnp.where(bits >= 0, bits, bits ^ jnp.int32(0x7FFFFFFF))
        keys_v[pl.ds(i * 16, 16)] = ks
        return jnp.maximum(m1, ks), jnp.maximum(m2, jnp.minimum(m1, ks))

    sentinel = jnp.full((16,), SIGN, jnp.int32)
    m1, m2 = _unrolled_fori(NV, 16, kt_body, (sentinel, sentinel))
    thr_lb = jnp.min(m2)

    def cp_body(i, selc):
        ks = keys_v[pl.ds(i * 16, 16)]
        m = ks >= thr_lb

        @pl.when(jnp.any(m))
        def _():
            mi = m.astype(jnp.int32)
            rank = selc + plsc.cumsum(mi) - mi
            plsc.store_scatter(comp_v, [rank], ks, mask=m)

        return selc + plsc.all_reduce_population_count(m)

    selc = _unrolled_fori(NV, 8, cp_body, jnp.zeros((16,), jnp.int32))
    csize = jnp.max(selc)
    nv2 = (csize + 15) // 16

    def count_subset(pred):
        def cnt(i, acc):
            ks = comp_v[pl.ds(i * 16, 16)]
            lanes_ok = (lax.iota(jnp.int32, 16) + i * 16) < csize
            return acc + (pred(ks) & lanes_ok).astype(jnp.int32)
        return jnp.sum(lax.fori_loop(0, nv2, cnt, jnp.zeros((16,), jnp.int32)))

    def bit_body(bi, prefix_b):
        cand_b = prefix_b | (jnp.int32(1) << (31 - bi))
        cand_s = cand_b ^ SIGN
        cnt = count_subset(lambda ks: ks >= cand_s)
        return jnp.where(cnt >= KEV, cand_b, prefix_b)

    prefix_b = lax.fori_loop(0, 32, bit_body, jnp.int32(0))
    thr = prefix_b ^ SIGN
    cnt_gt = count_subset(lambda ks: ks > thr)
    need = KEV - cnt_gt

    def fin_body(i, carries):
        selc, eqc = carries
        ks = keys_v[pl.ds(i * 16, 16)]
        t = lax.iota(jnp.int32, 16) + i * 16
        gt = ks > thr
        eq = ks == thr

        def with_ties():
            eqi = eq.astype(jnp.int32)
            eq_rank = eqc + plsc.cumsum(eqi) - eqi
            return gt | (eq & (eq_rank < need))

        sel = lax.cond(jnp.any(eq), with_ties, lambda: gt)
        m = sel | (lax.rem(t, 65) == 0)

        @pl.when(jnp.any(m))
        def _():
            mi = m.astype(jnp.int32)
            rank = selc + plsc.cumsum(mi) - mi
            plsc.store_scatter(chosen_v, [rank], t, mask=m & (rank < K))

        return (selc + plsc.all_reduce_population_count(m),
                eqc + plsc.all_reduce_population_count(eq))

    z16 = jnp.zeros((16,), jnp.int32)
    _unrolled_fori(NV, 8, fin_body, (z16, z16))

    pltpu.sync_copy(chosen_v, times_hbm.at[pl.ds(w * K, K)])
    for j in range(K // 16):
        idxf_v[pl.ds(j * 16, 16)] = chosen_v[pl.ds(j * 16, 16)] + w * T

    CH = 16
    seq_in = [pltpu.async_copy(seq_hbm.at[idxf_v.at[pl.ds(c * CH, CH)]],
                               row_seq.at[pl.ds(c * CH, CH)], gsems[c])
              for c in range(K // CH)]
    te_in = pltpu.async_copy(tt_hbm.at[chosen_v], row_te, gsems[K // CH])
    outs = []
    for c in range(K // CH):
        seq_in[c].wait()
        outs.append(pltpu.async_copy(row_seq.at[pl.ds(c * CH, CH)],
                                     gseq_hbm.at[w].at[pl.ds(c * CH, CH)],
                                     wsem))
    te_in.wait()
    outs.append(pltpu.async_copy(row_te, gte_hbm.at[w], wsem))
    for o in outs:
        o.wait()


def _sc_select_gather(event_scores, seq2, time_table):
    return pl.kernel(
        _sc_body,
        out_type=(
            jax.ShapeDtypeStruct((B * K,), jnp.int32),
            jax.ShapeDtypeStruct((B, K, D), jnp.float32),
            jax.ShapeDtypeStruct((B, K, D), jnp.float32),
        ),
        mesh=plsc.VectorSubcoreMesh(core_axis_name="c", subcore_axis_name="s",
                                    num_cores=2, num_subcores=16),
        compiler_params=pltpu.CompilerParams(needs_layout_passes=False),
        scratch_types=[
            pltpu.VMEM((T,), jnp.float32),
            pltpu.VMEM((T,), jnp.int32),
            pltpu.VMEM((T,), jnp.int32),
            pltpu.VMEM((K,), jnp.int32),
            pltpu.VMEM((K,), jnp.int32),
            pltpu.VMEM((K, D), jnp.float32),
            pltpu.VMEM((K, D), jnp.float32),
            [pltpu.SemaphoreType.DMA] * 5,
            pltpu.SemaphoreType.DMA,
        ],
    )(event_scores, seq2, time_table)


def _tc_body(times_ref, gseq_ref, hl_ref, gte_ref, wt_ref, b_ref,
             out_ref, hold_ref, hl_scr):
    g = pl.program_id(0)
    for j in range(BB):
        for r in range(K):
            idx = times_ref[g * BB * K + j * K + r]
            hl_scr[pl.ds(j * K + r, 1), :] = hl_ref[j, pl.ds(idx, 1), :]
    hl = hl_scr[...]
    hold_ref[...] = hl.reshape(BB, K, E)
    mx = jnp.max(hl, axis=-1, keepdims=True)
    ex = jnp.exp(hl - mx)
    sm = ex / jnp.sum(ex, axis=-1, keepdims=True)
    x = gseq_ref[...].reshape(BB * K, D)
    acc = jnp.dot(x, wt_ref[:D], preferred_element_type=jnp.float32)
    acc = acc + jnp.dot(sm, wt_ref[D:], preferred_element_type=jnp.float32)
    acc = acc + b_ref[...] + gte_ref[...].reshape(BB * K, D)
    out_ref[...] = acc.reshape(BB, K, D)


def _tc_project(times1, gseq, hl, gte, WT, b2):
    return pl.pallas_call(
        _tc_body,
        grid=(B // BB,),
        in_specs=[
            pl.BlockSpec(memory_space=pltpu.SMEM),
            pl.BlockSpec((BB, K, D), lambda b: (b, 0, 0)),
            pl.BlockSpec((BB, T, E), lambda b: (b, 0, 0)),
            pl.BlockSpec((BB, K, D), lambda b: (b, 0, 0)),
            pl.BlockSpec((D + E, D), lambda b: (0, 0)),
            pl.BlockSpec((1, D), lambda b: (0, 0)),
        ],
        out_specs=(
            pl.BlockSpec((BB, K, D), lambda b: (b, 0, 0)),
            pl.BlockSpec((BB, K, E), lambda b: (b, 0, 0)),
        ),
        out_shape=(
            jax.ShapeDtypeStruct((B, K, D), jnp.float32),
            jax.ShapeDtypeStruct((B, K, E), jnp.float32),
        ),
        scratch_shapes=[pltpu.VMEM((BB * K, E), jnp.float32)],
    )(times1, gseq, hl, gte, WT, b2)


def kernel(sequence, holder_logits, event_scores, W, b_lin, time_table):
    seq2 = sequence.reshape(B * T, D)
    times1, gseq, gte = _sc_select_gather(event_scores, seq2, time_table)
    entries, holders = _tc_project(times1, gseq, holder_logits, gte, W.T,
                                   b_lin.reshape(1, D))
    mask = jnp.ones((B, K), jnp.bool_)
    return entries, mask, times1.reshape(B, K), holders


# speedup vs baseline: 1.0137x; 1.0094x over previous
import jax
import jax.numpy as jnp
from jax import lax
from jax.experimental import pallas as pl
from jax.experimental.pallas import tpu as pltpu
from jax.experimental.pallas import tpu_sc as plsc

B, T, D, E = 32, 4096, 768, 64
K = 64
KEV = 32
NV = T // 16
SIGN = -2**31
BB = 4


def _unrolled_fori(n, unroll, body, carry):
    def outer(o, c):
        for u in range(unroll):
            c = body(o * unroll + u, c)
        return c
    return lax.fori_loop(0, n // unroll, outer, carry)


def _sc_body(ev_hbm, seq_hbm, tt_hbm,
             times_hbm, gseq_hbm, gte_hbm,
             raw_v, keys_v, comp_v, chosen_v, idxf_v,
             row_seq, row_te, sem1, sem2, sem3):
    w = lax.axis_index("s") * 2 + lax.axis_index("c")

    pltpu.sync_copy(ev_hbm.at[w], raw_v)

    def kt_body(i, car):
        m1, m2 = car
        v = raw_v[pl.ds(i * 16, 16)]
        bits = lax.bitcast_convert_type(v, jnp.int32)
        ks = jnp.where(bits >= 0, bits, bits ^ jnp.int32(0x7FFFFFFF))
        keys_v[pl.ds(i * 16, 16)] = ks
        return jnp.maximum(m1, ks), jnp.maximum(m2, jnp.minimum(m1, ks))

    sentinel = jnp.full((16,), SIGN, jnp.int32)
    m1, m2 = _unrolled_fori(NV, 16, kt_body, (sentinel, sentinel))
    thr_lb = jnp.min(m2)

    def cp_body(i, selc):
        ks = keys_v[pl.ds(i * 16, 16)]
        m = ks >= thr_lb
        mi = m.astype(jnp.int32)
        rank = selc + plsc.cumsum(mi) - mi
        plsc.store_scatter(comp_v, [rank], ks, mask=m)
        return selc + plsc.all_reduce_population_count(m)

    selc = _unrolled_fori(NV, 8, cp_body, jnp.zeros((16,), jnp.int32))
    csize = jnp.max(selc)
    nv2 = (csize + 15) // 16

    def count_subset(pred):
        def cnt(i, acc):
            ks = comp_v[pl.ds(i * 16, 16)]
            lanes_ok = (lax.iota(jnp.int32, 16) + i * 16) < csize
            return acc + (pred(ks) & lanes_ok).astype(jnp.int32)
        return jnp.sum(lax.fori_loop(0, nv2, cnt, jnp.zeros((16,), jnp.int32)))

    def bit_body(bi, prefix_b):
        cand_b = prefix_b | (jnp.int32(1) << (31 - bi))
        cand_s = cand_b ^ SIGN
        cnt = count_subset(lambda ks: ks >= cand_s)
        return jnp.where(cnt >= KEV, cand_b, prefix_b)

    prefix_b = lax.fori_loop(0, 32, bit_body, jnp.int32(0))
    thr = prefix_b ^ SIGN
    cnt_gt = count_subset(lambda ks: ks > thr)
    need = KEV - cnt_gt

    def fin_body(i, carries):
        selc, eqc = carries
        ks = keys_v[pl.ds(i * 16, 16)]
        t = lax.iota(jnp.int32, 16) + i * 16
        eq = ks == thr
        eqi = eq.astype(jnp.int32)
        eq_rank = eqc + plsc.cumsum(eqi) - eqi
        sel = (ks > thr) | (eq & (eq_rank < need))
        m = sel | (lax.rem(t, 65) == 0)
        mi = m.astype(jnp.int32)
        rank = selc + plsc.cumsum(mi) - mi
        plsc.store_scatter(chosen_v, [rank], t, mask=m & (rank < K))
        return (selc + plsc.all_reduce_population_count(m),
                eqc + plsc.all_reduce_population_count(eq))

    z16 = jnp.zeros((16,), jnp.int32)
    _unrolled_fori(NV, 8, fin_body, (z16, z16))

    pltpu.sync_copy(chosen_v, times_hbm.at[pl.ds(w * K, K)])
    for j in range(K // 16):
        idxf_v[pl.ds(j * 16, 16)] = chosen_v[pl.ds(j * 16, 16)] + w * T

    a_seq = pltpu.async_copy(seq_hbm.at[idxf_v], row_seq, sem1)
    a_tt = pltpu.async_copy(tt_hbm.at[chosen_v], row_te, sem2)
    a_seq.wait()
    w_seq = pltpu.async_copy(row_seq, gseq_hbm.at[w], sem3)
    a_tt.wait()
    w_te = pltpu.async_copy(row_te, gte_hbm.at[w], sem1)
    w_seq.wait()
    w_te.wait()


def _sc_select_gather(event_scores, seq2, time_table):
    return pl.kernel(
        _sc_body,
        out_type=(
            jax.ShapeDtypeStruct((B * K,), jnp.int32),
            jax.ShapeDtypeStruct((B, K, D), jnp.float32),
            jax.ShapeDtypeStruct((B, K, D), jnp.float32),
        ),
        mesh=plsc.VectorSubcoreMesh(core_axis_name="c", subcore_axis_name="s",
                                    num_cores=2, num_subcores=16),
        compiler_params=pltpu.CompilerParams(needs_layout_passes=False),
        scratch_types=[
            pltpu.VMEM((T,), jnp.float32),
            pltpu.VMEM((T,), jnp.int32),
            pltpu.VMEM((T,), jnp.int32),
            pltpu.VMEM((K,), jnp.int32),
            pltpu.VMEM((K,), jnp.int32),
            pltpu.VMEM((K, D), jnp.float32),
            pltpu.VMEM((K, D), jnp.float32),
            pltpu.SemaphoreType.DMA,
            pltpu.SemaphoreType.DMA,
            pltpu.SemaphoreType.DMA,
        ],
    )(event_scores, seq2, time_table)


def _tc_body(times_ref, gseq_ref, hl_ref, gte_ref, wt_ref, b_ref,
             out_ref, hold_ref, hl_scr):
    g = pl.program_id(0)
    for j in range(BB):
        for r in range(K):
            idx = times_ref[g * BB * K + j * K + r]
            hl_scr[pl.ds(j * K + r, 1), :] = hl_ref[j, pl.ds(idx, 1), :]
    hl = hl_scr[...]
    hold_ref[...] = hl.reshape(BB, K, E)
    mx = jnp.max(hl, axis=-1, keepdims=True)
    ex = jnp.exp(hl - mx)
    sm = ex / jnp.sum(ex, axis=-1, keepdims=True)
    x = gseq_ref[...].reshape(BB * K, D)
    acc = jnp.dot(x, wt_ref[:D], preferred_element_type=jnp.float32)
    acc = acc + jnp.dot(sm, wt_ref[D:], preferred_element_type=jnp.float32)
    acc = acc + b_ref[...] + gte_ref[...].reshape(BB * K, D)
    out_ref[...] = acc.reshape(BB, K, D)


def _tc_project(times1, gseq, hl, gte, WT, b2):
    return pl.pallas_call(
        _tc_body,
        grid=(B // BB,),
        in_specs=[
            pl.BlockSpec(memory_space=pltpu.SMEM),
            pl.BlockSpec((BB, K, D), lambda b: (b, 0, 0)),
            pl.BlockSpec((BB, T, E), lambda b: (b, 0, 0)),
            pl.BlockSpec((BB, K, D), lambda b: (b, 0, 0)),
            pl.BlockSpec((D + E, D), lambda b: (0, 0)),
            pl.BlockSpec((1, D), lambda b: (0, 0)),
        ],
        out_specs=(
            pl.BlockSpec((BB, K, D), lambda b: (b, 0, 0)),
            pl.BlockSpec((BB, K, E), lambda b: (b, 0, 0)),
        ),
        out_shape=(
            jax.ShapeDtypeStruct((B, K, D), jnp.float32),
            jax.ShapeDtypeStruct((B, K, E), jnp.float32),
        ),
        scratch_shapes=[pltpu.VMEM((BB * K, E), jnp.float32)],
    )(times1, gseq, hl, gte, WT, b2)


def kernel(sequence, holder_logits, event_scores, W, b_lin, time_table):
    seq2 = sequence.reshape(B * T, D)
    times1, gseq, gte = _sc_select_gather(event_scores, seq2, time_table)
    entries, holders = _tc_project(times1, gseq, holder_logits, gte, W.T,
                                   b_lin.reshape(1, D))
    mask = jnp.ones((B, K), jnp.bool_)
    return entries, mask, times1.reshape(B, K), holders
